# Initial kernel scaffold; baseline (speedup 1.0000x reference)
import jax
import jax.numpy as jnp
from jax.experimental import pallas as pl


def kernel(rpn_scores, rpn_coordinates, anchors, image_shape):
    raise NotImplementedError("write your pallas kernel here")


# speedup vs baseline: 24.7544x; 24.7544x over previous
import jax
import jax.numpy as jnp
from jax import lax
from jax.experimental import pallas as pl
from jax.experimental.pallas import tpu as pltpu
from jax.experimental.pallas import tpu_sc as plsc

N = 20000
NPAD = 20480
BS = 256
NB = NPAD // BS
NBV = NB // 16
BV = BS // 16
K = 300
KPAD = 304
THR = 0.7
NEG = -1.0e31
VTH = -1.0e30
EPS = 1e-9
L = 16
TILES = 16
CHUNK = NPAD // TILES


def _iota():
    return lax.iota(jnp.int32, L)


def _bcast_lane(vec, lane):
    return jnp.full((L,), jnp.max(jnp.where(_iota() == lane, vec, jnp.float32(NEG))))


def _nms_body(scores_hbm, ay1_hbm, ax1_hbm, ay2_hbm, ax2_hbm,
              dy_hbm, dx_hbm, dh_hbm, dw_hbm, hw_hbm, out_hbm,
              iay1, iax1, iay2, iax2, idy, idx_, idh, idw,
              y1l, x1l, y2l, x2l,
              spm_y1, spm_x1, spm_y2, spm_x2,
              scores_v, by1, bx1, by2, bx2, bm_v,
              acc_y1, acc_x1, acc_y2, acc_x2, acc_ar, outf, hw_v):
    c = lax.axis_index("c")
    s = lax.axis_index("s")

    @pl.when(c == 0)
    def _decode():
        base = s * CHUNK
        sl_in = pl.ds(base, CHUNK)
        pltpu.sync_copy(ay1_hbm.at[sl_in], iay1)
        pltpu.sync_copy(ax1_hbm.at[sl_in], iax1)
        pltpu.sync_copy(ay2_hbm.at[sl_in], iay2)
        pltpu.sync_copy(ax2_hbm.at[sl_in], iax2)
        pltpu.sync_copy(dy_hbm.at[sl_in], idy)
        pltpu.sync_copy(dx_hbm.at[sl_in], idx_)
        pltpu.sync_copy(dh_hbm.at[sl_in], idh)
        pltpu.sync_copy(dw_hbm.at[sl_in], idw)
        pltpu.sync_copy(hw_hbm, hw_v)
        hh = hw_v[pl.ds(0, L)]
        ww = hw_v[pl.ds(L, L)]

        def dloop(j, _):
            sl = pl.ds(j * L, L)
            ay1 = iay1[sl]
            ax1 = iax1[sl]
            ay2 = iay2[sl]
            ax2 = iax2[sl]
            dy = idy[sl]
            dx = idx_[sl]
            dh = idh[sl]
            dw = idw[sl]
            ah = ay2 - ay1
            aw = ax2 - ax1
            acy = ay1 + 0.5 * ah
            acx = ax1 + 0.5 * aw
            pcy = dy * ah + acy
            pcx = dx * aw + acx
            ph = jnp.exp(dh) * ah
            pw = jnp.exp(dw) * aw
            z = jnp.float32(0.0)
            y1l[sl] = jnp.minimum(jnp.maximum(pcy - 0.5 * ph, z), hh)
            x1l[sl] = jnp.minimum(jnp.maximum(pcx - 0.5 * pw, z), ww)
            y2l[sl] = jnp.minimum(jnp.maximum(pcy + 0.5 * ph, z), hh)
            x2l[sl] = jnp.minimum(jnp.maximum(pcx + 0.5 * pw, z), ww)
            return 0

        lax.fori_loop(0, CHUNK // L, dloop, 0)
        dst = pl.ds(base, CHUNK)
        pltpu.sync_copy(y1l, spm_y1.at[dst])
        pltpu.sync_copy(x1l, spm_x1.at[dst])
        pltpu.sync_copy(y2l, spm_y2.at[dst])
        pltpu.sync_copy(x2l, spm_x2.at[dst])

    plsc.subcore_barrier()

    @pl.when((c == 0) & (s == 0))
    def _nms():
        pltpu.sync_copy(scores_hbm, scores_v)
        pltpu.sync_copy(spm_y1, by1)
        pltpu.sync_copy(spm_x1, bx1)
        pltpu.sync_copy(spm_y2, by2)
        pltpu.sync_copy(spm_x2, bx2)
        iot = _iota()
        zf = jnp.zeros((L,), jnp.float32)
        negf = jnp.full((L,), NEG, jnp.float32)

        def zloop(k2, _):
            sl = pl.ds(k2 * L, L)
            acc_y1[sl] = zf
            acc_x1[sl] = zf
            acc_y2[sl] = zf
            acc_x2[sl] = zf
            acc_ar[sl] = zf
            return 0

        lax.fori_loop(0, KPAD // L, zloop, 0)

        def z2loop(k2, _):
            outf[pl.ds(k2 * L, L)] = zf
            return 0

        lax.fori_loop(0, (KPAD * 4) // L, z2loop, 0)

        def bmloop(g, _):
            def one(t, acc):
                b = g * L + t

                def inner(j, mv):
                    return jnp.maximum(mv, scores_v[pl.ds(b * BS + j * L, L)])

                mv = lax.fori_loop(0, BV, inner, negf)
                return jnp.where(iot == t, jnp.max(mv), acc)

            bm_v[pl.ds(g * L, L)] = lax.fori_loop(0, L, one, zf)
            return 0

        lax.fori_loop(0, NBV, bmloop, 0)

        BIGI = jnp.int32(2 ** 30)

        def cond(carry):
            cnt, alive = carry
            return (cnt < K) & (alive > 0)

        def body(carry):
            cnt, _ = carry

            def tloop(g, cr):
                mv, bv = cr
                v = bm_v[pl.ds(g * L, L)]
                upd = v > mv
                return (jnp.where(upd, v, mv), jnp.where(upd, g * L + iot, bv))

            mv, bv = lax.fori_loop(0, NBV, tloop, (negf, jnp.zeros((L,), jnp.int32)))
            m = jnp.max(mv)
            b = jnp.min(jnp.where(mv == m, bv, BIGI))
            alive = (m > VTH).astype(jnp.int32)

            def floop(j, cr):
                rmv, riv = cr
                v = scores_v[pl.ds(b * BS + j * L, L)]
                upd = v > rmv
                return (jnp.where(upd, v, rmv), jnp.where(upd, j * L + iot, riv))

            rmv, riv = lax.fori_loop(0, BV, floop, (negf, jnp.zeros((L,), jnp.int32)))
            m2 = jnp.max(rmv)
            ib = jnp.min(jnp.where(rmv == m2, riv, BIGI))
            gidx = b * BS + ib

            q = gidx // L
            r = gidx - q * L
            ksl = pl.ds(q * L, L)
            scores_v[ksl] = jnp.where(iot == r, negf, scores_v[ksl])

            def rloop(j, mvv):
                return jnp.maximum(mvv, scores_v[pl.ds(b * BS + j * L, L)])

            nmv = lax.fori_loop(0, BV, rloop, negf)
            qb = b // L
            rb = b - qb * L
            bsl = pl.ds(qb * L, L)
            bm_v[bsl] = jnp.where(iot == rb, jnp.full((L,), jnp.max(nmv)), bm_v[bsl])

            cy1 = _bcast_lane(by1[ksl], r)
            cx1 = _bcast_lane(bx1[ksl], r)
            cy2 = _bcast_lane(by2[ksl], r)
            cx2 = _bcast_lane(bx2[ksl], r)
            car = jnp.maximum(cy2 - cy1, 0.0) * jnp.maximum(cx2 - cx1, 0.0)

            nv = (cnt + (L - 1)) // L

            def chk(k2, mx):
                sl = pl.ds(k2 * L, L)
                iy1 = jnp.maximum(acc_y1[sl], cy1)
                ix1 = jnp.maximum(acc_x1[sl], cx1)
                iy2 = jnp.minimum(acc_y2[sl], cy2)
                ix2 = jnp.minimum(acc_x2[sl], cx2)
                inter = jnp.maximum(iy2 - iy1, 0.0) * jnp.maximum(ix2 - ix1, 0.0)
                iou = inter / (acc_ar[sl] + car - inter + EPS)
                return jnp.maximum(mx, iou)

            mx = lax.fori_loop(0, nv, chk, zf)
            accept = (jnp.max(mx) <= THR) & (alive > 0)

            @pl.when(accept)
            def _acc():
                qa = cnt // L
                ra = cnt - qa * L
                asl = pl.ds(qa * L, L)
                sel = iot == ra
                acc_y1[asl] = jnp.where(sel, cy1, acc_y1[asl])
                acc_x1[asl] = jnp.where(sel, cx1, acc_x1[asl])
                acc_y2[asl] = jnp.where(sel, cy2, acc_y2[asl])
                acc_x2[asl] = jnp.where(sel, cx2, acc_x2[asl])
                acc_ar[asl] = jnp.where(sel, car, acc_ar[asl])
                fo = 4 * cnt
                qo = fo // L
                ro = fo - qo * L
                osl = pl.ds(qo * L, L)
                row = jnp.where(iot == ro, cy1,
                                jnp.where(iot == ro + 1, cx1,
                                          jnp.where(iot == ro + 2, cy2,
                                                    jnp.where(iot == ro + 3, cx2,
                                                              outf[osl]))))
                outf[osl] = row

            return (cnt + accept.astype(jnp.int32), alive)

        lax.while_loop(cond, body, (jnp.int32(0), jnp.int32(1)))
        pltpu.sync_copy(outf.at[pl.ds(0, 4 * K)], out_hbm)


_mesh = plsc.VectorSubcoreMesh(core_axis_name="c", subcore_axis_name="s")

_nms_call = pl.kernel(
    _nms_body,
    out_type=jax.ShapeDtypeStruct((4 * K,), jnp.float32),
    mesh=_mesh,
    compiler_params=pltpu.CompilerParams(needs_layout_passes=False),
    scratch_types=(
        [pltpu.VMEM((CHUNK,), jnp.float32)] * 8 +
        [pltpu.VMEM((CHUNK,), jnp.float32)] * 4 +
        [pltpu.VMEM_SHARED((NPAD,), jnp.float32)] * 4 +
        [pltpu.VMEM((NPAD,), jnp.float32)] * 5 +
        [pltpu.VMEM((NB,), jnp.float32)] +
        [pltpu.VMEM((KPAD,), jnp.float32)] * 5 +
        [pltpu.VMEM((KPAD * 4,), jnp.float32)] +
        [pltpu.VMEM((2 * L,), jnp.float32)]
    ),
)


@jax.jit
def kernel(rpn_scores, rpn_coordinates, anchors, image_shape):
    img_h = image_shape[0].astype(jnp.float32)
    img_w = image_shape[1].astype(jnp.float32)
    hw = jnp.concatenate([jnp.full((L,), img_h), jnp.full((L,), img_w)])
    padf = jnp.zeros((NPAD - N,), jnp.float32)
    scores_p = jnp.concatenate(
        [rpn_scores.astype(jnp.float32), jnp.full((NPAD - N,), NEG, jnp.float32)])
    a = anchors.astype(jnp.float32)
    d = rpn_coordinates.astype(jnp.float32)
    cols = [jnp.concatenate([a[:, i], padf]) for i in range(4)] + \
           [jnp.concatenate([d[:, i], padf]) for i in range(4)]
    out = _nms_call(scores_p, *cols, hw)
    return lax.stop_gradient(out.reshape(K, 4))


# speedup vs baseline: 29.1485x; 1.1775x over previous
import jax
import jax.numpy as jnp
from jax import lax
from jax.experimental import pallas as pl
from jax.experimental.pallas import tpu as pltpu
from jax.experimental.pallas import tpu_sc as plsc

N = 20000
NPAD = 20480
BS = 80
NB = NPAD // BS
NG = NB // 16
BV = BS // 16
K = 300
KPAD = 320
THR = 0.7
NEG = -1.0e31
VTH = -1.0e30
EPS = 1e-9
L = 16
TILES = 16
CHUNK = NPAD // TILES
NBT = CHUNK // BS


def _iota():
    return lax.iota(jnp.int32, L)


def _nms_body(scores_hbm, ay1_hbm, ax1_hbm, ay2_hbm, ax2_hbm,
              dy_hbm, dx_hbm, dh_hbm, dw_hbm, hw_hbm, out_hbm,
              iay1, iax1, iay2, iax2, idy, idx_, idh, idw,
              y1l, x1l, y2l, x2l, scl, bml,
              spm_y1, spm_x1, spm_y2, spm_x2, spm_bm,
              scores_v, by1, bx1, by2, bx2, bm_v, bm2_v,
              acc_y1, acc_x1, acc_y2, acc_x2, acc_ar, outf, hw_v,
              sem_s, sem_1, sem_2, sem_3, sem_4, sem_5):
    c = lax.axis_index("c")
    s = lax.axis_index("s")
    iot = _iota()
    zf = jnp.zeros((L,), jnp.float32)
    negf = jnp.full((L,), NEG, jnp.float32)
    is0 = (c == 0) & (s == 0)

    @pl.when(is0)
    def _start_scores():
        pltpu.async_copy(scores_hbm, scores_v, sem_s)

    @pl.when(c == 0)
    def _decode():
        base = s * CHUNK
        sl_in = pl.ds(base, CHUNK)
        pltpu.sync_copy(ay1_hbm.at[sl_in], iay1)
        pltpu.sync_copy(ax1_hbm.at[sl_in], iax1)
        pltpu.sync_copy(ay2_hbm.at[sl_in], iay2)
        pltpu.sync_copy(ax2_hbm.at[sl_in], iax2)
        pltpu.sync_copy(dy_hbm.at[sl_in], idy)
        pltpu.sync_copy(dx_hbm.at[sl_in], idx_)
        pltpu.sync_copy(dh_hbm.at[sl_in], idh)
        pltpu.sync_copy(dw_hbm.at[sl_in], idw)
        pltpu.sync_copy(scores_hbm.at[sl_in], scl)
        pltpu.sync_copy(hw_hbm, hw_v)
        hh = hw_v[pl.ds(0, L)]
        ww = hw_v[pl.ds(L, L)]

        def dloop(j, _):
            sl = pl.ds(j * L, L)
            ay1 = iay1[sl]
            ax1 = iax1[sl]
            ay2 = iay2[sl]
            ax2 = iax2[sl]
            dy = idy[sl]
            dx = idx_[sl]
            dh = idh[sl]
            dw = idw[sl]
            ah = ay2 - ay1
            aw = ax2 - ax1
            acy = ay1 + 0.5 * ah
            acx = ax1 + 0.5 * aw
            pcy = dy * ah + acy
            pcx = dx * aw + acx
            ph = jnp.exp(dh) * ah
            pw = jnp.exp(dw) * aw
            z = jnp.float32(0.0)
            y1l[sl] = jnp.minimum(jnp.maximum(pcy - 0.5 * ph, z), hh)
            x1l[sl] = jnp.minimum(jnp.maximum(pcx - 0.5 * pw, z), ww)
            y2l[sl] = jnp.minimum(jnp.maximum(pcy + 0.5 * ph, z), hh)
            x2l[sl] = jnp.minimum(jnp.maximum(pcx + 0.5 * pw, z), ww)
            return 0

        lax.fori_loop(0, CHUNK // L, dloop, 0)

        def bloop(t, acc):
            def inner(j, mv):
                return jnp.maximum(mv, scl[pl.ds(t * BS + j * L, L)])

            mv = lax.fori_loop(0, BV, inner, negf)
            return jnp.where(iot == t, jnp.max(mv), acc)

        bml[pl.ds(0, L)] = lax.fori_loop(0, NBT, bloop, negf)

        dst = pl.ds(base, CHUNK)
        pltpu.sync_copy(y1l, spm_y1.at[dst])
        pltpu.sync_copy(x1l, spm_x1.at[dst])
        pltpu.sync_copy(y2l, spm_y2.at[dst])
        pltpu.sync_copy(x2l, spm_x2.at[dst])
        pltpu.sync_copy(bml, spm_bm.at[pl.ds(s * L, L)])

    plsc.subcore_barrier()

    @pl.when(is0)
    def _nms():
        cp1 = pltpu.async_copy(spm_y1, by1, sem_1)
        cp2 = pltpu.async_copy(spm_x1, bx1, sem_2)
        cp3 = pltpu.async_copy(spm_y2, by2, sem_3)
        cp4 = pltpu.async_copy(spm_x2, bx2, sem_4)
        cp5 = pltpu.async_copy(spm_bm, bm_v, sem_5)

        def zloop(k2, _):
            sl = pl.ds(k2 * L, L)
            acc_y1[sl] = zf
            acc_x1[sl] = zf
            acc_y2[sl] = zf
            acc_x2[sl] = zf
            acc_ar[sl] = zf
            return 0

        lax.fori_loop(0, KPAD // L, zloop, 0)

        def z2loop(k2, _):
            outf[pl.ds(k2 * L, L)] = zf
            return 0

        lax.fori_loop(0, (KPAD * 4) // L, z2loop, 0)

        cp5.wait()

        def gloop(g, acc):
            return jnp.where(iot == g, jnp.max(bm_v[pl.ds(g * L, L)]), acc)

        bm2_v[pl.ds(0, L)] = lax.fori_loop(0, NG, gloop, negf)

        cp1.wait()
        cp2.wait()
        cp3.wait()
        cp4.wait()
        pltpu.make_async_copy(scores_hbm, scores_v, sem_s).wait()

        BIGI = jnp.int32(2 ** 30)

        def cond(carry):
            cnt, alive = carry
            return (cnt < K) & (alive > 0)

        def body(carry):
            cnt, _ = carry

            g2 = bm2_v[pl.ds(0, L)]
            m = jnp.max(g2)
            alive = (m > VTH).astype(jnp.int32)
            g = jnp.min(jnp.where(g2 == m, iot, BIGI))
            gsl = pl.ds(g * L, L)
            chunk = bm_v[gsl]
            b = g * L + jnp.min(jnp.where(chunk == m, iot, BIGI))

            def floop(j, cr):
                rmv, smv, riv = cr
                v = scores_v[pl.ds(b * BS + j * L, L)]
                upd = v > rmv
                smv = jnp.maximum(smv, jnp.where(upd, rmv, v))
                return (jnp.where(upd, v, rmv), smv,
                        jnp.where(upd, j * L + iot, riv))

            rmv, smv, riv = lax.fori_loop(
                0, BV, floop, (negf, negf, jnp.zeros((L,), jnp.int32)))
            ib = jnp.min(jnp.where(rmv == m, riv, BIGI))
            gidx = b * BS + ib
            q = gidx // L
            r = gidx - q * L

            ksl = pl.ds(q * L, L)
            kv = scores_v[ksl]
            scores_v[ksl] = jnp.where(iot == r, negf, kv)

            nbm = jnp.max(jnp.where(iot == r, smv, rmv))
            nchunk = jnp.where(iot == (b - g * L), nbm, chunk)
            bm_v[gsl] = nchunk
            bm2_v[pl.ds(0, L)] = jnp.where(iot == g, jnp.max(nchunk), g2)

            cy1 = jnp.full((L,), jnp.max(jnp.where(iot == r, by1[ksl], negf)))
            cx1 = jnp.full((L,), jnp.max(jnp.where(iot == r, bx1[ksl], negf)))
            cy2 = jnp.full((L,), jnp.max(jnp.where(iot == r, by2[ksl], negf)))
            cx2 = jnp.full((L,), jnp.max(jnp.where(iot == r, bx2[ksl], negf)))
            car = jnp.maximum(cy2 - cy1, 0.0) * jnp.maximum(cx2 - cx1, 0.0)

            def one(ofs, mx):
                sl = pl.ds(ofs, L)
                iy1 = jnp.maximum(acc_y1[sl], cy1)
                ix1 = jnp.maximum(acc_x1[sl], cx1)
                iy2 = jnp.minimum(acc_y2[sl], cy2)
                ix2 = jnp.minimum(acc_x2[sl], cx2)
                inter = jnp.maximum(iy2 - iy1, 0.0) * jnp.maximum(ix2 - ix1, 0.0)
                return jnp.maximum(mx, inter / (acc_ar[sl] + car - inter + EPS))

            def chk(k4, accs):
                a0, a1, a2, a3 = accs
                base = k4 * (4 * L)
                return (one(base, a0), one(base + L, a1),
                        one(base + 2 * L, a2), one(base + 3 * L, a3))

            n4 = (cnt + (4 * L - 1)) // (4 * L)
            a0, a1, a2, a3 = lax.fori_loop(0, n4, chk, (zf, zf, zf, zf))
            mxv = jnp.maximum(jnp.maximum(a0, a1), jnp.maximum(a2, a3))
            accept = (jnp.max(mxv) <= THR) & (alive > 0)

            @pl.when(accept)
            def _acc():
                qa = cnt // L
                ra = cnt - qa * L
                asl = pl.ds(qa * L, L)
                sel = iot == ra
                acc_y1[asl] = jnp.where(sel, cy1, acc_y1[asl])
                acc_x1[asl] = jnp.where(sel, cx1, acc_x1[asl])
                acc_y2[asl] = jnp.where(sel, cy2, acc_y2[asl])
                acc_x2[asl] = jnp.where(sel, cx2, acc_x2[asl])
                acc_ar[asl] = jnp.where(sel, car, acc_ar[asl])
                fo = 4 * cnt
                qo = fo // L
                ro = fo - qo * L
                osl = pl.ds(qo * L, L)
                outf[osl] = jnp.where(iot == ro, cy1,
                                      jnp.where(iot == ro + 1, cx1,
                                                jnp.where(iot == ro + 2, cy2,
                                                          jnp.where(iot == ro + 3, cx2,
                                                                    outf[osl]))))

            return (cnt + accept.astype(jnp.int32), alive)

        lax.while_loop(cond, body, (jnp.int32(0), jnp.int32(1)))
        pltpu.sync_copy(outf.at[pl.ds(0, 4 * K)], out_hbm)


_mesh = plsc.VectorSubcoreMesh(core_axis_name="c", subcore_axis_name="s")

_nms_call = pl.kernel(
    _nms_body,
    out_type=jax.ShapeDtypeStruct((4 * K,), jnp.float32),
    mesh=_mesh,
    compiler_params=pltpu.CompilerParams(needs_layout_passes=False),
    scratch_types=(
        [pltpu.VMEM((CHUNK,), jnp.float32)] * 8 +
        [pltpu.VMEM((CHUNK,), jnp.float32)] * 4 +
        [pltpu.VMEM((CHUNK,), jnp.float32)] +
        [pltpu.VMEM((L,), jnp.float32)] +
        [pltpu.VMEM_SHARED((NPAD,), jnp.float32)] * 4 +
        [pltpu.VMEM_SHARED((NB,), jnp.float32)] +
        [pltpu.VMEM((NPAD,), jnp.float32)] * 5 +
        [pltpu.VMEM((NB,), jnp.float32)] +
        [pltpu.VMEM((L,), jnp.float32)] +
        [pltpu.VMEM((KPAD,), jnp.float32)] * 5 +
        [pltpu.VMEM((KPAD * 4,), jnp.float32)] +
        [pltpu.VMEM((2 * L,), jnp.float32)] +
        [pltpu.SemaphoreType.DMA] * 6
    ),
)


@jax.jit
def kernel(rpn_scores, rpn_coordinates, anchors, image_shape):
    img_h = image_shape[0].astype(jnp.float32)
    img_w = image_shape[1].astype(jnp.float32)
    hw = jnp.concatenate([jnp.full((L,), img_h), jnp.full((L,), img_w)])
    padf = jnp.zeros((NPAD - N,), jnp.float32)
    scores_p = jnp.concatenate(
        [rpn_scores.astype(jnp.float32), jnp.full((NPAD - N,), NEG, jnp.float32)])
    a = anchors.astype(jnp.float32)
    d = rpn_coordinates.astype(jnp.float32)
    cols = [jnp.concatenate([a[:, i], padf]) for i in range(4)] + \
           [jnp.concatenate([d[:, i], padf]) for i in range(4)]
    out = _nms_call(scores_p, *cols, hw)
    return lax.stop_gradient(out.reshape(K, 4))
